# baseline (speedup 1.0000x reference)
import functools

import jax
import jax.numpy as jnp
from jax.experimental import pallas as pl

_RESO = 32
_GRID = 2 * _RESO ** 3


def _stage0_body(pts_ref, wp_ref, bp_ref, w0_ref, b0_ref, w1_ref, b1_ref,
                 ws_ref, out_ref):
    h = jnp.dot(pts_ref[...], wp_ref[...],
                preferred_element_type=jnp.float32) + bp_ref[...]
    n0 = jnp.dot(jnp.maximum(h, 0.0), w0_ref[...],
                 preferred_element_type=jnp.float32) + b0_ref[...]
    dx = jnp.dot(jnp.maximum(n0, 0.0), w1_ref[...],
                 preferred_element_type=jnp.float32) + b1_ref[...]
    out_ref[...] = jnp.dot(h, ws_ref[...],
                           preferred_element_type=jnp.float32) + dx


def _block_body(last, net_ref, pool_ref, invc_ref, w0a_ref, w0b_ref, b0_ref,
                w1_ref, b1_ref, wsa_ref, wsb_ref, wc_ref, bc_ref, out_ref):
    net = net_ref[...]
    pooled = pool_ref[...].astype(jnp.float32) * invc_ref[...]
    h = (jnp.dot(jnp.maximum(net, 0.0), w0a_ref[...],
                 preferred_element_type=jnp.float32)
         + jnp.dot(jnp.maximum(pooled, 0.0), w0b_ref[...],
                   preferred_element_type=jnp.float32)
         + b0_ref[...])
    dx = jnp.dot(jnp.maximum(h, 0.0), w1_ref[...],
                 preferred_element_type=jnp.float32) + b1_ref[...]
    out = (jnp.dot(net, wsa_ref[...], preferred_element_type=jnp.float32)
           + jnp.dot(pooled, wsb_ref[...], preferred_element_type=jnp.float32)
           + dx)
    if last:
        out = jnp.dot(out, wc_ref[...],
                      preferred_element_type=jnp.float32) + bc_ref[...]
    out_ref[...] = out


_TILE = 2000


def _full(shape):
    return pl.BlockSpec(shape, lambda i: (0,) * len(shape))


def _stage0(pts, wp, bp, w0, b0, w1, b1, ws):
    m = pts.shape[0]
    grid = (m // _TILE,)
    return pl.pallas_call(
        _stage0_body,
        grid=grid,
        in_specs=[
            pl.BlockSpec((_TILE, 3), lambda i: (i, 0)),
            _full(wp.shape), _full(bp.shape), _full(w0.shape), _full(b0.shape),
            _full(w1.shape), _full(b1.shape), _full(ws.shape),
        ],
        out_specs=pl.BlockSpec((_TILE, 128), lambda i: (i, 0)),
        out_shape=jax.ShapeDtypeStruct((m, 128), jnp.float32),
    )(pts, wp, bp, w0, b0, w1, b1, ws)


def _block(last, net, pooled, invc, w0a, w0b, b0, w1, b1, wsa, wsb, wc, bc):
    m = net.shape[0]
    grid = (m // _TILE,)
    return pl.pallas_call(
        functools.partial(_block_body, last),
        grid=grid,
        in_specs=[
            pl.BlockSpec((_TILE, 128), lambda i: (i, 0)),
            pl.BlockSpec((_TILE, 128), lambda i: (i, 0)),
            pl.BlockSpec((_TILE, 1), lambda i: (i, 0)),
            _full(w0a.shape), _full(w0b.shape), _full(b0.shape),
            _full(w1.shape), _full(b1.shape),
            _full(wsa.shape), _full(wsb.shape),
            _full(wc.shape), _full(bc.shape),
        ],
        out_specs=pl.BlockSpec((_TILE, 128), lambda i: (i, 0)),
        out_shape=jax.ShapeDtypeStruct((m, 128), jnp.float32),
    )(net, pooled, invc, w0a, w0b, b0, w1, b1, wsa, wsb, wc, bc)


def kernel(points, index_grid, fc_pos_W, fc_pos_b, blocks_fc0_W, blocks_fc0_b,
           blocks_fc1_W, blocks_fc1_b, blocks_sc_W, fc_c_W, fc_c_b):
    b, n, _ = points.shape
    cd = fc_c_W.shape[0]
    m = b * n
    nseg = b * _GRID

    seg = (index_grid.reshape(b, n)
           + jnp.arange(b, dtype=jnp.int32)[:, None] * _GRID).reshape(-1)

    cnt = jax.ops.segment_sum(jnp.ones((m,), jnp.float32), seg,
                              num_segments=nseg)
    inv_cnt = 1.0 / jnp.maximum(cnt, 1.0)
    invc_pts = inv_cnt[seg].reshape(m, 1)

    pts = points.reshape(m, 3)
    bias2 = lambda v: v.reshape(1, -1)

    net = _stage0(pts, fc_pos_W.T, bias2(fc_pos_b),
                  blocks_fc0_W[0].T, bias2(blocks_fc0_b[0]),
                  blocks_fc1_W[0].T, bias2(blocks_fc1_b[0]),
                  blocks_sc_W[0].T)

    nb = blocks_fc0_W.shape[0]
    for i in range(1, nb):
        sums = jax.ops.segment_sum(net.astype(jnp.bfloat16), seg,
                                   num_segments=nseg)
        pooled = sums[seg]
        w0 = blocks_fc0_W[i].T
        ws = blocks_sc_W[i].T
        net = _block(i == nb - 1, net, pooled, invc_pts,
                     w0[:128], w0[128:], bias2(blocks_fc0_b[i]),
                     blocks_fc1_W[i].T, bias2(blocks_fc1_b[i]),
                     ws[:128], ws[128:], fc_c_W.T, bias2(fc_c_b))

    sums_c = jax.ops.segment_sum(net, seg, num_segments=nseg)
    fea = sums_c * inv_cnt[:, None]
    fea = fea.reshape(b, _GRID, cd).transpose(0, 2, 1)
    return fea.reshape(b, 2 * cd, _RESO, _RESO, _RESO)


# speedup vs baseline: 1.6806x; 1.6806x over previous
import functools

import jax
import jax.numpy as jnp
from jax.experimental import pallas as pl

_RESO = 32
_GRID = 2 * _RESO ** 3


def _stage0_body(pts_ref, wp_ref, bp_ref, w0_ref, b0_ref, w1_ref, b1_ref,
                 ws_ref, out_ref):
    h = jnp.dot(pts_ref[...], wp_ref[...],
                preferred_element_type=jnp.float32) + bp_ref[...]
    n0 = jnp.dot(jnp.maximum(h, 0.0), w0_ref[...],
                 preferred_element_type=jnp.float32) + b0_ref[...]
    dx = jnp.dot(jnp.maximum(n0, 0.0), w1_ref[...],
                 preferred_element_type=jnp.float32) + b1_ref[...]
    out_ref[...] = jnp.dot(h, ws_ref[...],
                           preferred_element_type=jnp.float32) + dx


def _block_body(last, net_ref, pool_ref, invc_ref, w0a_ref, w0b_ref, b0_ref,
                w1_ref, b1_ref, wsa_ref, wsb_ref, wc_ref, bc_ref, out_ref):
    net = net_ref[...]
    pooled = pool_ref[...] * invc_ref[...]
    h = (jnp.dot(jnp.maximum(net, 0.0), w0a_ref[...],
                 preferred_element_type=jnp.float32)
         + jnp.dot(jnp.maximum(pooled, 0.0), w0b_ref[...],
                   preferred_element_type=jnp.float32)
         + b0_ref[...])
    dx = jnp.dot(jnp.maximum(h, 0.0), w1_ref[...],
                 preferred_element_type=jnp.float32) + b1_ref[...]
    out = (jnp.dot(net, wsa_ref[...], preferred_element_type=jnp.float32)
           + jnp.dot(pooled, wsb_ref[...], preferred_element_type=jnp.float32)
           + dx)
    if last:
        out = jnp.dot(out, wc_ref[...],
                      preferred_element_type=jnp.float32) + bc_ref[...]
    out_ref[...] = out


def _finalize_body(sums_ref, invc_ref, out_ref):
    x = sums_ref[...] * invc_ref[...]
    out_ref[...] = jnp.swapaxes(x, 1, 2)


_CHUNK = 2048


def _finalize(sums_c, inv_cnt, b, g, cd):
    sums3 = sums_c.reshape(b, g, cd)
    invc3 = inv_cnt.reshape(b, g, 1)
    return pl.pallas_call(
        _finalize_body,
        grid=(b, g // _CHUNK),
        in_specs=[
            pl.BlockSpec((1, _CHUNK, cd), lambda i, j: (i, j, 0)),
            pl.BlockSpec((1, _CHUNK, 1), lambda i, j: (i, j, 0)),
        ],
        out_specs=pl.BlockSpec((1, cd, _CHUNK), lambda i, j: (i, 0, j)),
        out_shape=jax.ShapeDtypeStruct((b, cd, g), jnp.float32),
    )(sums3, invc3)


_TILE = 2000


def _full(shape):
    return pl.BlockSpec(shape, lambda i: (0,) * len(shape))


def _stage0(pts, wp, bp, w0, b0, w1, b1, ws):
    m = pts.shape[0]
    grid = (m // _TILE,)
    return pl.pallas_call(
        _stage0_body,
        grid=grid,
        in_specs=[
            pl.BlockSpec((_TILE, 3), lambda i: (i, 0)),
            _full(wp.shape), _full(bp.shape), _full(w0.shape), _full(b0.shape),
            _full(w1.shape), _full(b1.shape), _full(ws.shape),
        ],
        out_specs=pl.BlockSpec((_TILE, 128), lambda i: (i, 0)),
        out_shape=jax.ShapeDtypeStruct((m, 128), jnp.float32),
    )(pts, wp, bp, w0, b0, w1, b1, ws)


def _block(last, net, pooled, invc, w0a, w0b, b0, w1, b1, wsa, wsb, wc, bc):
    m = net.shape[0]
    grid = (m // _TILE,)
    return pl.pallas_call(
        functools.partial(_block_body, last),
        grid=grid,
        in_specs=[
            pl.BlockSpec((_TILE, 128), lambda i: (i, 0)),
            pl.BlockSpec((_TILE, 128), lambda i: (i, 0)),
            pl.BlockSpec((_TILE, 1), lambda i: (i, 0)),
            _full(w0a.shape), _full(w0b.shape), _full(b0.shape),
            _full(w1.shape), _full(b1.shape),
            _full(wsa.shape), _full(wsb.shape),
            _full(wc.shape), _full(bc.shape),
        ],
        out_specs=pl.BlockSpec((_TILE, 128), lambda i: (i, 0)),
        out_shape=jax.ShapeDtypeStruct((m, 128), jnp.float32),
    )(net, pooled, invc, w0a, w0b, b0, w1, b1, wsa, wsb, wc, bc)


def kernel(points, index_grid, fc_pos_W, fc_pos_b, blocks_fc0_W, blocks_fc0_b,
           blocks_fc1_W, blocks_fc1_b, blocks_sc_W, fc_c_W, fc_c_b):
    b, n, _ = points.shape
    cd = fc_c_W.shape[0]
    m = b * n
    nseg = b * _GRID

    seg = (index_grid.reshape(b, n)
           + jnp.arange(b, dtype=jnp.int32)[:, None] * _GRID).reshape(-1)

    cnt = jax.ops.segment_sum(jnp.ones((m,), jnp.float32), seg,
                              num_segments=nseg)
    inv_cnt = 1.0 / jnp.maximum(cnt, 1.0)
    invc_pts = inv_cnt[seg].reshape(m, 1)

    pts = points.reshape(m, 3)
    bias2 = lambda v: v.reshape(1, -1)

    net = _stage0(pts, fc_pos_W.T, bias2(fc_pos_b),
                  blocks_fc0_W[0].T, bias2(blocks_fc0_b[0]),
                  blocks_fc1_W[0].T, bias2(blocks_fc1_b[0]),
                  blocks_sc_W[0].T)

    nb = blocks_fc0_W.shape[0]
    for i in range(1, nb):
        sums = jax.ops.segment_sum(net, seg, num_segments=nseg)
        pooled = sums[seg]
        w0 = blocks_fc0_W[i].T
        ws = blocks_sc_W[i].T
        net = _block(i == nb - 1, net, pooled, invc_pts,
                     w0[:128], w0[128:], bias2(blocks_fc0_b[i]),
                     blocks_fc1_W[i].T, bias2(blocks_fc1_b[i]),
                     ws[:128], ws[128:], fc_c_W.T, bias2(fc_c_b))

    sums_c = jax.ops.segment_sum(net, seg, num_segments=nseg)
    fea = _finalize(sums_c, inv_cnt, b, _GRID, cd)
    return fea.reshape(b, 2 * cd, _RESO, _RESO, _RESO)


# speedup vs baseline: 1.7212x; 1.0242x over previous
import functools

import jax
import jax.numpy as jnp
from jax.experimental import pallas as pl

_RESO = 32
_GRID = 2 * _RESO ** 3


def _stage0_body(pts_ref, wp_ref, bp_ref, w0_ref, b0_ref, w1_ref, b1_ref,
                 ws_ref, out_ref):
    h = jnp.dot(pts_ref[...], wp_ref[...],
                preferred_element_type=jnp.float32) + bp_ref[...]
    n0 = jnp.dot(jnp.maximum(h, 0.0), w0_ref[...],
                 preferred_element_type=jnp.float32) + b0_ref[...]
    dx = jnp.dot(jnp.maximum(n0, 0.0), w1_ref[...],
                 preferred_element_type=jnp.float32) + b1_ref[...]
    out_ref[...] = jnp.dot(h, ws_ref[...],
                           preferred_element_type=jnp.float32) + dx


def _block_body(last, net_ref, pool_ref, invc_ref, w0a_ref, w0b_ref, b0_ref,
                w1_ref, b1_ref, wsa_ref, wsb_ref, wc_ref, bc_ref, out_ref):
    net = net_ref[...]
    pooled = pool_ref[...] * invc_ref[...]
    h = (jnp.dot(jnp.maximum(net, 0.0), w0a_ref[...],
                 preferred_element_type=jnp.float32)
         + jnp.dot(jnp.maximum(pooled, 0.0), w0b_ref[...],
                   preferred_element_type=jnp.float32)
         + b0_ref[...])
    dx = jnp.dot(jnp.maximum(h, 0.0), w1_ref[...],
                 preferred_element_type=jnp.float32) + b1_ref[...]
    out = (jnp.dot(net, wsa_ref[...], preferred_element_type=jnp.float32)
           + jnp.dot(pooled, wsb_ref[...], preferred_element_type=jnp.float32)
           + dx)
    if last:
        out = jnp.dot(out, wc_ref[...],
                      preferred_element_type=jnp.float32) + bc_ref[...]
    out_ref[...] = out


def _finalize_body(sums_ref, invc_ref, out_ref):
    x = sums_ref[...] * invc_ref[...]
    out_ref[...] = jnp.swapaxes(x, 1, 2)


_CHUNK = 2048


def _finalize(sums_c, inv_cnt, b, g, cd):
    sums3 = sums_c.reshape(b, g, cd)
    invc3 = inv_cnt.reshape(b, g, 1)
    return pl.pallas_call(
        _finalize_body,
        grid=(b, g // _CHUNK),
        in_specs=[
            pl.BlockSpec((1, _CHUNK, cd), lambda i, j: (i, j, 0)),
            pl.BlockSpec((1, _CHUNK, 1), lambda i, j: (i, j, 0)),
        ],
        out_specs=pl.BlockSpec((1, cd, _CHUNK), lambda i, j: (i, 0, j)),
        out_shape=jax.ShapeDtypeStruct((b, cd, g), jnp.float32),
    )(sums3, invc3)


_TILE = 4000


def _full(shape):
    return pl.BlockSpec(shape, lambda i: (0,) * len(shape))


def _stage0(pts, wp, bp, w0, b0, w1, b1, ws):
    m = pts.shape[0]
    grid = (m // _TILE,)
    return pl.pallas_call(
        _stage0_body,
        grid=grid,
        in_specs=[
            pl.BlockSpec((_TILE, 3), lambda i: (i, 0)),
            _full(wp.shape), _full(bp.shape), _full(w0.shape), _full(b0.shape),
            _full(w1.shape), _full(b1.shape), _full(ws.shape),
        ],
        out_specs=pl.BlockSpec((_TILE, 128), lambda i: (i, 0)),
        out_shape=jax.ShapeDtypeStruct((m, 128), jnp.float32),
    )(pts, wp, bp, w0, b0, w1, b1, ws)


def _block(last, net, pooled, invc, w0a, w0b, b0, w1, b1, wsa, wsb, wc, bc):
    m = net.shape[0]
    grid = (m // _TILE,)
    return pl.pallas_call(
        functools.partial(_block_body, last),
        grid=grid,
        in_specs=[
            pl.BlockSpec((_TILE, 128), lambda i: (i, 0)),
            pl.BlockSpec((_TILE, 128), lambda i: (i, 0)),
            pl.BlockSpec((_TILE, 1), lambda i: (i, 0)),
            _full(w0a.shape), _full(w0b.shape), _full(b0.shape),
            _full(w1.shape), _full(b1.shape),
            _full(wsa.shape), _full(wsb.shape),
            _full(wc.shape), _full(bc.shape),
        ],
        out_specs=pl.BlockSpec((_TILE, 128), lambda i: (i, 0)),
        out_shape=jax.ShapeDtypeStruct((m, 128), jnp.float32),
    )(net, pooled, invc, w0a, w0b, b0, w1, b1, wsa, wsb, wc, bc)


def kernel(points, index_grid, fc_pos_W, fc_pos_b, blocks_fc0_W, blocks_fc0_b,
           blocks_fc1_W, blocks_fc1_b, blocks_sc_W, fc_c_W, fc_c_b):
    b, n, _ = points.shape
    cd = fc_c_W.shape[0]
    m = b * n
    nseg = b * _GRID

    seg = (index_grid.reshape(b, n)
           + jnp.arange(b, dtype=jnp.int32)[:, None] * _GRID).reshape(-1)

    cnt = jax.ops.segment_sum(jnp.ones((m,), jnp.float32), seg,
                              num_segments=nseg)
    inv_cnt = 1.0 / jnp.maximum(cnt, 1.0)
    invc_pts = inv_cnt[seg].reshape(m, 1)

    pts = points.reshape(m, 3)
    bias2 = lambda v: v.reshape(1, -1)

    net = _stage0(pts, fc_pos_W.T, bias2(fc_pos_b),
                  blocks_fc0_W[0].T, bias2(blocks_fc0_b[0]),
                  blocks_fc1_W[0].T, bias2(blocks_fc1_b[0]),
                  blocks_sc_W[0].T)

    nb = blocks_fc0_W.shape[0]
    for i in range(1, nb):
        sums = jax.ops.segment_sum(net, seg, num_segments=nseg)
        pooled = sums[seg]
        w0 = blocks_fc0_W[i].T
        ws = blocks_sc_W[i].T
        net = _block(i == nb - 1, net, pooled, invc_pts,
                     w0[:128], w0[128:], bias2(blocks_fc0_b[i]),
                     blocks_fc1_W[i].T, bias2(blocks_fc1_b[i]),
                     ws[:128], ws[128:], fc_c_W.T, bias2(fc_c_b))

    sums_c = jax.ops.segment_sum(net, seg, num_segments=nseg)
    fea = _finalize(sums_c, inv_cnt, b, _GRID, cd)
    return fea.reshape(b, 2 * cd, _RESO, _RESO, _RESO)


# speedup vs baseline: 1.7384x; 1.0100x over previous
import functools

import jax
import jax.numpy as jnp
from jax.experimental import pallas as pl

_RESO = 32
_GRID = 2 * _RESO ** 3


def _stage0_body(pts_ref, wp_ref, bp_ref, w0_ref, b0_ref, w1_ref, b1_ref,
                 ws_ref, out_ref):
    h = jnp.dot(pts_ref[...], wp_ref[...],
                preferred_element_type=jnp.float32) + bp_ref[...]
    n0 = jnp.dot(jnp.maximum(h, 0.0), w0_ref[...],
                 preferred_element_type=jnp.float32) + b0_ref[...]
    dx = jnp.dot(jnp.maximum(n0, 0.0), w1_ref[...],
                 preferred_element_type=jnp.float32) + b1_ref[...]
    out_ref[...] = jnp.dot(h, ws_ref[...],
                           preferred_element_type=jnp.float32) + dx


def _block_body(last, net_ref, pool_ref, invc_ref, w0a_ref, w0b_ref, b0_ref,
                w1_ref, b1_ref, wsa_ref, wsb_ref, wc_ref, bc_ref, out_ref):
    net = net_ref[...]
    pooled = pool_ref[...] * invc_ref[...]
    h = (jnp.dot(jnp.maximum(net, 0.0), w0a_ref[...],
                 preferred_element_type=jnp.float32)
         + jnp.dot(jnp.maximum(pooled, 0.0), w0b_ref[...],
                   preferred_element_type=jnp.float32)
         + b0_ref[...])
    dx = jnp.dot(jnp.maximum(h, 0.0), w1_ref[...],
                 preferred_element_type=jnp.float32) + b1_ref[...]
    out = (jnp.dot(net, wsa_ref[...], preferred_element_type=jnp.float32)
           + jnp.dot(pooled, wsb_ref[...], preferred_element_type=jnp.float32)
           + dx)
    if last:
        out = jnp.dot(out, wc_ref[...],
                      preferred_element_type=jnp.float32) + bc_ref[...]
    out_ref[...] = out


def _finalize_body(sums_ref, invc_ref, out_ref):
    x = sums_ref[...] * invc_ref[...]
    out_ref[...] = jnp.swapaxes(x, 1, 2)


_CHUNK = 2048


def _finalize(sums_c, inv_cnt, b, g, cd):
    sums3 = sums_c.reshape(b, g, cd)
    invc3 = inv_cnt.reshape(b, g, 1)
    return pl.pallas_call(
        _finalize_body,
        grid=(b, g // _CHUNK),
        in_specs=[
            pl.BlockSpec((1, _CHUNK, cd), lambda i, j: (i, j, 0)),
            pl.BlockSpec((1, _CHUNK, 1), lambda i, j: (i, j, 0)),
        ],
        out_specs=pl.BlockSpec((1, cd, _CHUNK), lambda i, j: (i, 0, j)),
        out_shape=jax.ShapeDtypeStruct((b, cd, g), jnp.float32),
    )(sums3, invc3)


_TILE = 10000


def _full(shape):
    return pl.BlockSpec(shape, lambda i: (0,) * len(shape))


def _stage0(pts, wp, bp, w0, b0, w1, b1, ws):
    m = pts.shape[0]
    grid = (m // _TILE,)
    return pl.pallas_call(
        _stage0_body,
        grid=grid,
        in_specs=[
            pl.BlockSpec((_TILE, 3), lambda i: (i, 0)),
            _full(wp.shape), _full(bp.shape), _full(w0.shape), _full(b0.shape),
            _full(w1.shape), _full(b1.shape), _full(ws.shape),
        ],
        out_specs=pl.BlockSpec((_TILE, 128), lambda i: (i, 0)),
        out_shape=jax.ShapeDtypeStruct((m, 128), jnp.float32),
    )(pts, wp, bp, w0, b0, w1, b1, ws)


def _block(last, net, pooled, invc, w0a, w0b, b0, w1, b1, wsa, wsb, wc, bc):
    m = net.shape[0]
    grid = (m // _TILE,)
    return pl.pallas_call(
        functools.partial(_block_body, last),
        grid=grid,
        in_specs=[
            pl.BlockSpec((_TILE, 128), lambda i: (i, 0)),
            pl.BlockSpec((_TILE, 128), lambda i: (i, 0)),
            pl.BlockSpec((_TILE, 1), lambda i: (i, 0)),
            _full(w0a.shape), _full(w0b.shape), _full(b0.shape),
            _full(w1.shape), _full(b1.shape),
            _full(wsa.shape), _full(wsb.shape),
            _full(wc.shape), _full(bc.shape),
        ],
        out_specs=pl.BlockSpec((_TILE, 128), lambda i: (i, 0)),
        out_shape=jax.ShapeDtypeStruct((m, 128), jnp.float32),
    )(net, pooled, invc, w0a, w0b, b0, w1, b1, wsa, wsb, wc, bc)


def kernel(points, index_grid, fc_pos_W, fc_pos_b, blocks_fc0_W, blocks_fc0_b,
           blocks_fc1_W, blocks_fc1_b, blocks_sc_W, fc_c_W, fc_c_b):
    b, n, _ = points.shape
    cd = fc_c_W.shape[0]
    m = b * n
    nseg = b * _GRID

    seg = (index_grid.reshape(b, n)
           + jnp.arange(b, dtype=jnp.int32)[:, None] * _GRID).reshape(-1)

    cnt = jax.ops.segment_sum(jnp.ones((m,), jnp.float32), seg,
                              num_segments=nseg)
    inv_cnt = 1.0 / jnp.maximum(cnt, 1.0)
    invc_pts = inv_cnt[seg].reshape(m, 1)

    pts = points.reshape(m, 3)
    bias2 = lambda v: v.reshape(1, -1)

    net = _stage0(pts, fc_pos_W.T, bias2(fc_pos_b),
                  blocks_fc0_W[0].T, bias2(blocks_fc0_b[0]),
                  blocks_fc1_W[0].T, bias2(blocks_fc1_b[0]),
                  blocks_sc_W[0].T)

    nb = blocks_fc0_W.shape[0]
    for i in range(1, nb):
        sums = jax.ops.segment_sum(net, seg, num_segments=nseg)
        pooled = sums[seg]
        w0 = blocks_fc0_W[i].T
        ws = blocks_sc_W[i].T
        net = _block(i == nb - 1, net, pooled, invc_pts,
                     w0[:128], w0[128:], bias2(blocks_fc0_b[i]),
                     blocks_fc1_W[i].T, bias2(blocks_fc1_b[i]),
                     ws[:128], ws[128:], fc_c_W.T, bias2(fc_c_b))

    sums_c = jax.ops.segment_sum(net, seg, num_segments=nseg)
    fea = _finalize(sums_c, inv_cnt, b, _GRID, cd)
    return fea.reshape(b, 2 * cd, _RESO, _RESO, _RESO)
